# baseline (speedup 1.0000x reference)
import functools

import jax
import jax.numpy as jnp
from jax import lax
from jax.experimental import pallas as pl
from jax.experimental.pallas import tpu as pltpu
from jax.experimental.pallas import tpu_sc as plsc

_INFO = plsc.get_sparse_core_info()
_NC = _INFO.num_cores
_NS = _INFO.num_subcores
_NW = _NC * _NS

_CHUNK = 128


def _gather_body(n_chunks, d, table_hbm, idx_hbm, out_hbm, idx_v, rows_v, sem):
    wid = lax.axis_index("s") * _NC + lax.axis_index("c")
    row_base = wid * n_chunks
    pltpu.sync_copy(idx_hbm.at[pl.ds(row_base, n_chunks)], idx_v)

    def chunk(g, _):
        pltpu.async_copy(table_hbm.at[idx_v.at[g]], rows_v, sem).wait()
        pltpu.sync_copy(
            rows_v, out_hbm.at[pl.ds((row_base + g) * _CHUNK, _CHUNK)]
        )
        return _

    lax.fori_loop(0, n_chunks, chunk, 0)


def kernel(x, table):
    b0, b1 = x.shape
    n, d = table.shape
    total = b0 * b1
    assert total % (_NW * _CHUNK) == 0
    n_chunks = total // (_NW * _CHUNK)

    idx2d = x.reshape(_NW * n_chunks, _CHUNK)

    mesh = plsc.VectorSubcoreMesh(core_axis_name="c", subcore_axis_name="s")
    run = pl.kernel(
        functools.partial(_gather_body, n_chunks, d),
        out_type=jax.ShapeDtypeStruct((total, d), table.dtype),
        mesh=mesh,
        scratch_types=[
            pltpu.VMEM((n_chunks, _CHUNK), jnp.int32),
            pltpu.VMEM((_CHUNK, d), table.dtype),
            pltpu.SemaphoreType.DMA,
        ],
        compiler_params=pltpu.CompilerParams(use_tc_tiling_on_sc=False),
    )
    out = run(table, idx2d)
    return out.reshape(b0, b1, d)


# speedup vs baseline: 1.1153x; 1.1153x over previous
import functools

import jax
import jax.numpy as jnp
from jax import lax
from jax.experimental import pallas as pl
from jax.experimental.pallas import tpu as pltpu
from jax.experimental.pallas import tpu_sc as plsc

_INFO = plsc.get_sparse_core_info()
_NC = _INFO.num_cores
_NS = _INFO.num_subcores
_NW = _NC * _NS

_CHUNK = 128
_NBUF = 4


def _gather_body(n_chunks, d, table_hbm, idx_hbm, out_hbm, idx_v, rows_v,
                 gsem, wsem):
    wid = lax.axis_index("s") * _NC + lax.axis_index("c")
    row_base = wid * n_chunks
    pltpu.sync_copy(idx_hbm.at[pl.ds(row_base, n_chunks)], idx_v)

    def start_gather(g, b):
        pltpu.async_copy(table_hbm.at[idx_v.at[g]], rows_v.at[b], gsem.at[b])

    def wait_gather(g, b):
        pltpu.make_async_copy(
            table_hbm.at[idx_v.at[g]], rows_v.at[b], gsem.at[b]
        ).wait()

    def out_slice(g):
        return out_hbm.at[pl.ds((row_base + g) * _CHUNK, _CHUNK)]

    def start_write(g, b):
        pltpu.async_copy(rows_v.at[b], out_slice(g), wsem.at[b])

    def wait_write(g, b):
        pltpu.make_async_copy(rows_v.at[b], out_slice(g), wsem.at[b]).wait()

    for b in range(_NBUF):
        start_gather(b, b)

    n_outer = n_chunks // _NBUF

    def ring_round(go, issue_next):
        for b in range(_NBUF):
            g = go * _NBUF + b
            wait_gather(g, b)
            start_write(g, b)
            wait_write(g, b)
            if issue_next:
                start_gather(g + _NBUF, b)

    lax.fori_loop(
        0, n_outer - 1, lambda go, c: (ring_round(go, True), c)[1], 0
    )
    ring_round(n_outer - 1, False)


def kernel(x, table):
    b0, b1 = x.shape
    n, d = table.shape
    total = b0 * b1
    assert total % (_NW * _CHUNK) == 0
    n_chunks = total // (_NW * _CHUNK)

    idx2d = x.reshape(_NW * n_chunks, _CHUNK)

    mesh = plsc.VectorSubcoreMesh(core_axis_name="c", subcore_axis_name="s")
    run = pl.kernel(
        functools.partial(_gather_body, n_chunks, d),
        out_type=jax.ShapeDtypeStruct((total, d), table.dtype),
        mesh=mesh,
        scratch_types=[
            pltpu.VMEM((n_chunks, _CHUNK), jnp.int32),
            pltpu.VMEM((_NBUF, _CHUNK, d), table.dtype),
            pltpu.SemaphoreType.DMA((_NBUF,)),
            pltpu.SemaphoreType.DMA((_NBUF,)),
        ],
        compiler_params=pltpu.CompilerParams(use_tc_tiling_on_sc=False),
    )
    out = run(table, idx2d)
    return out.reshape(b0, b1, d)


# speedup vs baseline: 1.1161x; 1.0008x over previous
import functools

import jax
import jax.numpy as jnp
from jax import lax
from jax.experimental import pallas as pl
from jax.experimental.pallas import tpu as pltpu
from jax.experimental.pallas import tpu_sc as plsc

_INFO = plsc.get_sparse_core_info()
_NC = _INFO.num_cores
_NS = _INFO.num_subcores
_NW = _NC * _NS

_CHUNK = 128
_NBUF = 8
_LEAD = 4


def _gather_body(n_chunks, d, table_hbm, idx_hbm, out_hbm, idx_v, rows_v,
                 gsem, wsem):
    wid = lax.axis_index("s") * _NC + lax.axis_index("c")
    row_base = wid * n_chunks
    pltpu.sync_copy(idx_hbm.at[pl.ds(row_base, n_chunks)], idx_v)

    def start_gather(g, b):
        pltpu.async_copy(table_hbm.at[idx_v.at[g]], rows_v.at[b], gsem.at[b])

    def wait_gather(g, b):
        pltpu.make_async_copy(
            table_hbm.at[idx_v.at[g]], rows_v.at[b], gsem.at[b]
        ).wait()

    def out_slice(g):
        return out_hbm.at[pl.ds((row_base + g) * _CHUNK, _CHUNK)]

    def start_write(g, b):
        pltpu.async_copy(rows_v.at[b], out_slice(g), wsem.at[b])

    def wait_write(g, b):
        pltpu.make_async_copy(rows_v.at[b], out_slice(g), wsem.at[b]).wait()

    for g0 in range(_LEAD):
        start_gather(g0, g0)

    def chunk_iter(g, carry):
        b = lax.rem(g, _NBUF)
        wait_gather(g, b)
        start_write(g, b)
        nxt = g + _LEAD
        bw = lax.rem(nxt, _NBUF)

        @pl.when(nxt >= _NBUF)
        def _():
            wait_write(nxt - _NBUF, bw)

        @pl.when(nxt < n_chunks)
        def _():
            start_gather(nxt, bw)

        return carry

    lax.fori_loop(0, n_chunks, chunk_iter, 0)

    for g0 in range(n_chunks - (_NBUF - _LEAD), n_chunks):
        wait_write(g0, g0 % _NBUF)


def kernel(x, table):
    b0, b1 = x.shape
    n, d = table.shape
    total = b0 * b1
    assert total % (_NW * _CHUNK) == 0
    n_chunks = total // (_NW * _CHUNK)

    idx2d = x.reshape(_NW * n_chunks, _CHUNK)

    mesh = plsc.VectorSubcoreMesh(core_axis_name="c", subcore_axis_name="s")
    run = pl.kernel(
        functools.partial(_gather_body, n_chunks, d),
        out_type=jax.ShapeDtypeStruct((total, d), table.dtype),
        mesh=mesh,
        scratch_types=[
            pltpu.VMEM((n_chunks, _CHUNK), jnp.int32),
            pltpu.VMEM((_NBUF, _CHUNK, d), table.dtype),
            pltpu.SemaphoreType.DMA((_NBUF,)),
            pltpu.SemaphoreType.DMA((_NBUF,)),
        ],
        compiler_params=pltpu.CompilerParams(use_tc_tiling_on_sc=False),
    )
    out = run(table, idx2d)
    return out.reshape(b0, b1, d)


# speedup vs baseline: 1.1171x; 1.0009x over previous
import functools

import jax
import jax.numpy as jnp
from jax import lax
from jax.experimental import pallas as pl
from jax.experimental.pallas import tpu as pltpu
from jax.experimental.pallas import tpu_sc as plsc

_INFO = plsc.get_sparse_core_info()
_NC = _INFO.num_cores
_NS = _INFO.num_subcores
_NW = _NC * _NS

_CHUNK = 256
_NBUF = 4
_LEAD = 2


def _gather_body(n_chunks, d, table_hbm, idx_hbm, out_hbm, idx_v, rows_v,
                 gsem, wsem):
    wid = lax.axis_index("s") * _NC + lax.axis_index("c")
    row_base = wid * n_chunks
    pltpu.sync_copy(idx_hbm.at[pl.ds(row_base, n_chunks)], idx_v)

    def start_gather(g, b):
        pltpu.async_copy(table_hbm.at[idx_v.at[g]], rows_v.at[b], gsem.at[b])

    def wait_gather(g, b):
        pltpu.make_async_copy(
            table_hbm.at[idx_v.at[g]], rows_v.at[b], gsem.at[b]
        ).wait()

    def out_slice(g):
        return out_hbm.at[pl.ds((row_base + g) * _CHUNK, _CHUNK)]

    def start_write(g, b):
        pltpu.async_copy(rows_v.at[b], out_slice(g), wsem.at[b])

    def wait_write(g, b):
        pltpu.make_async_copy(rows_v.at[b], out_slice(g), wsem.at[b]).wait()

    for g0 in range(_LEAD):
        start_gather(g0, g0)

    def chunk_iter(g, carry):
        b = lax.rem(g, _NBUF)
        wait_gather(g, b)
        start_write(g, b)
        nxt = g + _LEAD
        bw = lax.rem(nxt, _NBUF)

        @pl.when(nxt >= _NBUF)
        def _():
            wait_write(nxt - _NBUF, bw)

        @pl.when(nxt < n_chunks)
        def _():
            start_gather(nxt, bw)

        return carry

    lax.fori_loop(0, n_chunks, chunk_iter, 0)

    for g0 in range(n_chunks - (_NBUF - _LEAD), n_chunks):
        wait_write(g0, g0 % _NBUF)


def kernel(x, table):
    b0, b1 = x.shape
    n, d = table.shape
    total = b0 * b1
    assert total % (_NW * _CHUNK) == 0
    n_chunks = total // (_NW * _CHUNK)

    idx2d = x.reshape(_NW * n_chunks, _CHUNK)

    mesh = plsc.VectorSubcoreMesh(core_axis_name="c", subcore_axis_name="s")
    run = pl.kernel(
        functools.partial(_gather_body, n_chunks, d),
        out_type=jax.ShapeDtypeStruct((total, d), table.dtype),
        mesh=mesh,
        scratch_types=[
            pltpu.VMEM((n_chunks, _CHUNK), jnp.int32),
            pltpu.VMEM((_NBUF, _CHUNK, d), table.dtype),
            pltpu.SemaphoreType.DMA((_NBUF,)),
            pltpu.SemaphoreType.DMA((_NBUF,)),
        ],
        compiler_params=pltpu.CompilerParams(use_tc_tiling_on_sc=False),
    )
    out = run(table, idx2d)
    return out.reshape(b0, b1, d)
